# baseline (speedup 1.0000x reference)
import functools

import jax
import jax.numpy as jnp
from jax import lax
from jax.experimental import pallas as pl
from jax.experimental.pallas import tpu as pltpu
from jax.experimental.pallas import tpu_sc as plsc

N = 10000
P = 10240
DI = 128
DO = 64
E = 320000
NC = 2
NS = 16
NTILES = NC * NS
CHUNK = 128
CPT0 = 128
CPT1 = 32
EP = NS * (CPT0 + CPT1) * CHUNK
ROWS0 = NS * CPT0
SENT = N
CNT_ROWS = 256
RPT = P // NS
ZROWS = 160
NBUF = 4

_sc_mesh = plsc.VectorSubcoreMesh(
    core_axis_name="c", subcore_axis_name="s", num_cores=NC, num_subcores=NS
)


def _deg_body(src_hbm, dst_hbm, out_hbm, idx_s, idx_d, cnt_s, cnt_d):
    c = lax.axis_index("c")
    s = lax.axis_index("s")
    tile = c * NS + s
    nch = jnp.where(c == 0, CPT0, CPT1)
    base = jnp.where(c == 0, s * CPT0, ROWS0 + s * CPT1)
    zero16 = jnp.zeros((16,), jnp.float32)
    ones16 = jnp.ones((16,), jnp.float32)

    def zloop(r, carry):
        for cc in range(8):
            cnt_s[r, pl.ds(cc * 16, 16)] = zero16
            cnt_d[r, pl.ds(cc * 16, 16)] = zero16
        return carry

    lax.fori_loop(0, 128, zloop, 0)

    def lloop(i, carry):
        pltpu.sync_copy(src_hbm.at[pl.ds(base + i * 32, 32)],
                        idx_s.at[pl.ds(i * 32, 32)])
        pltpu.sync_copy(dst_hbm.at[pl.ds(base + i * 32, 32)],
                        idx_d.at[pl.ds(i * 32, 32)])
        return carry

    lax.fori_loop(0, nch // 32, lloop, 0)

    def cloop(r, carry):
        for cc in range(8):
            v = idx_s[r, pl.ds(cc * 16, 16)]
            plsc.addupdate_scatter(cnt_s, [v >> 7, v & 127], ones16)
            w = idx_d[r, pl.ds(cc * 16, 16)]
            plsc.addupdate_scatter(cnt_d, [w >> 7, w & 127], ones16)
        return carry

    lax.fori_loop(0, nch, cloop, 0)

    pltpu.sync_copy(cnt_s, out_hbm.at[0, tile])
    pltpu.sync_copy(cnt_d, out_hbm.at[1, tile])


_deg = pl.kernel(
    _deg_body,
    out_type=jax.ShapeDtypeStruct((2, NTILES, 128, 128), jnp.float32),
    mesh=_sc_mesh,
    scratch_types=[
        pltpu.VMEM((CPT0, CHUNK), jnp.int32),
        pltpu.VMEM((CPT0, CHUNK), jnp.int32),
        pltpu.VMEM((128, 128), jnp.float32),
        pltpu.VMEM((128, 128), jnp.float32),
    ],
    compiler_params=pltpu.CompilerParams(needs_layout_passes=False),
)


CPTT = EP // CHUNK // NS
PH = 2
CPP = CPTT // PH
NGP = CPP // NBUF


def _prop_body(h_hbm, src_hbm, dst_hbm, zer_hbm, out_hbm, idx_s, idx_d, rows,
               acc, gsa, gsb, ssa):
    c = lax.axis_index("c")
    s = lax.axis_index("s")

    @pl.when(c == 0)
    def _():
        for k in range(RPT // ZROWS):
            pltpu.sync_copy(zer_hbm, acc.at[pl.ds(s * RPT + k * ZROWS, ZROWS)])
        plsc.subcore_barrier()

        def fire_g(cb, slot0, sem):
            for b in range(NBUF):
                pltpu.async_copy(h_hbm.at[idx_s.at[cb + b]],
                                 rows.at[slot0 + b], sem)

        def fire_s(cb, slot0, sem):
            for b in range(NBUF):
                pltpu.async_copy(rows.at[slot0 + b],
                                 acc.at[idx_d.at[cb + b]], sem, add=True)

        def drain_rows(sem):
            for b in range(NBUF):
                pltpu.make_async_copy(h_hbm.at[pl.ds(0, CHUNK)],
                                      rows.at[b], sem).wait()

        for ph in range(PH):
            base = s * CPTT + ph * CPP
            pltpu.sync_copy(src_hbm.at[pl.ds(base, CPP)], idx_s)
            pltpu.sync_copy(dst_hbm.at[pl.ds(base, CPP)], idx_d)
            fire_g(0, 0, gsa)

            def body(k, carry):
                cb = k * 2 * NBUF
                fire_g(cb + NBUF, NBUF, gsb)
                drain_rows(gsa)
                fire_s(cb, 0, ssa)
                drain_rows(ssa)
                fire_g(cb + 2 * NBUF, 0, gsa)
                drain_rows(gsb)
                fire_s(cb + NBUF, NBUF, ssa)
                drain_rows(ssa)
                return carry

            lax.fori_loop(0, NGP // 2 - 1, body, 0)
            cb = (NGP - 2) * NBUF
            fire_g(cb + NBUF, NBUF, gsb)
            drain_rows(gsa)
            fire_s(cb, 0, ssa)
            drain_rows(ssa)
            drain_rows(gsb)
            fire_s(cb + NBUF, NBUF, ssa)
            drain_rows(ssa)

        plsc.subcore_barrier()
        pltpu.sync_copy(acc.at[pl.ds(s * RPT, RPT)],
                        out_hbm.at[pl.ds(s * RPT, RPT)])


_prop = pl.kernel(
    _prop_body,
    out_type=jax.ShapeDtypeStruct((P, DO), jnp.float32),
    mesh=_sc_mesh,
    scratch_types=[
        pltpu.VMEM((CPP, CHUNK), jnp.int32),
        pltpu.VMEM((CPP, CHUNK), jnp.int32),
        pltpu.VMEM((2 * NBUF, CHUNK, DO), jnp.float32),
        pltpu.VMEM_SHARED((P, DO), jnp.float32),
        pltpu.SemaphoreType.DMA,
        pltpu.SemaphoreType.DMA,
        pltpu.SemaphoreType.DMA,
    ],
    compiler_params=pltpu.CompilerParams(
        needs_layout_passes=False, use_tc_tiling_on_sc=False
    ),
)

BR = 1280
G = P // BR


def _k1_body(x_ref, w_ref, cs_ref, cd_ref, h0_ref, ns_ref, nd_ref):
    cs = jnp.sum(cs_ref[...], axis=0)
    cd = jnp.sum(cd_ref[...], axis=0)
    ns = lax.rsqrt(jnp.maximum(cs, 1.0))
    nd = lax.rsqrt(jnp.maximum(cd, 1.0))
    ns_ref[0, :] = ns
    nd_ref[0, :] = nd
    h0_ref[...] = jnp.dot(x_ref[...], w_ref[...],
                          preferred_element_type=jnp.float32) * ns[:, None]


_k1 = pl.pallas_call(
    _k1_body,
    grid=(G,),
    in_specs=[
        pl.BlockSpec((BR, DI), lambda i: (i, 0)),
        pl.BlockSpec((DI, DO), lambda i: (0, 0)),
        pl.BlockSpec((NTILES, BR), lambda i: (0, i)),
        pl.BlockSpec((NTILES, BR), lambda i: (0, i)),
    ],
    out_specs=[
        pl.BlockSpec((BR, DO), lambda i: (i, 0)),
        pl.BlockSpec((1, BR), lambda i: (0, i)),
        pl.BlockSpec((1, BR), lambda i: (0, i)),
    ],
    out_shape=[
        jax.ShapeDtypeStruct((P, DO), jnp.float32),
        jax.ShapeDtypeStruct((1, P), jnp.float32),
        jax.ShapeDtypeStruct((1, P), jnp.float32),
    ],
)


def _k2_body(p_ref, ns_ref, nd_ref, w_ref, b_ref, out_ref):
    agg = p_ref[...] * nd_ref[0, :][:, None]
    h = jnp.tanh(agg + b_ref[0, :][None, :])
    out_ref[...] = jnp.dot(h, w_ref[...],
                           preferred_element_type=jnp.float32) * ns_ref[0, :][:, None]


_k2 = pl.pallas_call(
    _k2_body,
    grid=(G,),
    in_specs=[
        pl.BlockSpec((BR, DO), lambda i: (i, 0)),
        pl.BlockSpec((1, BR), lambda i: (0, i)),
        pl.BlockSpec((1, BR), lambda i: (0, i)),
        pl.BlockSpec((DO, DO), lambda i: (0, 0)),
        pl.BlockSpec((1, DO), lambda i: (0, 0)),
    ],
    out_specs=pl.BlockSpec((BR, DO), lambda i: (i, 0)),
    out_shape=jax.ShapeDtypeStruct((P, DO), jnp.float32),
)


def _k3_body(p_ref, ns_ref, nd_ref, b_ref, out_ref):
    agg = p_ref[...] * nd_ref[0, :][:, None]
    out_ref[...] = jnp.tanh(agg + b_ref[0, :][None, :]) * ns_ref[0, :][:, None]


_k3 = pl.pallas_call(
    _k3_body,
    grid=(G,),
    in_specs=[
        pl.BlockSpec((BR, DO), lambda i: (i, 0)),
        pl.BlockSpec((1, BR), lambda i: (0, i)),
        pl.BlockSpec((1, BR), lambda i: (0, i)),
        pl.BlockSpec((1, DO), lambda i: (0, 0)),
    ],
    out_specs=pl.BlockSpec((BR, DO), lambda i: (i, 0)),
    out_shape=jax.ShapeDtypeStruct((P, DO), jnp.float32),
)


def _k4_body(p_ref, nd_ref, wmu_ref, bmu_ref, wls_ref, bls_ref, eps_ref,
             mu_ref, sg_ref, z_ref):
    agg = p_ref[...] * nd_ref[0, :][:, None]
    mu = jnp.tanh(jnp.dot(agg, wmu_ref[...],
                          preferred_element_type=jnp.float32) + bmu_ref[0, :][None, :])
    ls = jnp.tanh(jnp.dot(agg, wls_ref[...],
                          preferred_element_type=jnp.float32) + bls_ref[0, :][None, :])
    sg = jnp.exp(ls)
    mu_ref[...] = mu
    sg_ref[...] = sg
    z_ref[...] = mu + sg * eps_ref[...]


_k4 = pl.pallas_call(
    _k4_body,
    grid=(G,),
    in_specs=[
        pl.BlockSpec((BR, DO), lambda i: (i, 0)),
        pl.BlockSpec((1, BR), lambda i: (0, i)),
        pl.BlockSpec((DO, DO), lambda i: (0, 0)),
        pl.BlockSpec((1, DO), lambda i: (0, 0)),
        pl.BlockSpec((DO, DO), lambda i: (0, 0)),
        pl.BlockSpec((1, DO), lambda i: (0, 0)),
        pl.BlockSpec((BR, DO), lambda i: (i, 0)),
    ],
    out_specs=[
        pl.BlockSpec((BR, DO), lambda i: (i, 0)),
        pl.BlockSpec((BR, DO), lambda i: (i, 0)),
        pl.BlockSpec((BR, DO), lambda i: (i, 0)),
    ],
    out_shape=[
        jax.ShapeDtypeStruct((P, DO), jnp.float32),
        jax.ShapeDtypeStruct((P, DO), jnp.float32),
        jax.ShapeDtypeStruct((P, DO), jnp.float32),
    ],
)


def kernel(features, edge_index, W1, b1, W2, b2, Wmu, bmu, Wls, bls, eps):
    src = edge_index[0]
    dst = edge_index[1]
    pad_src = jnp.full((EP - E,), SENT, jnp.int32)
    pad_dst = SENT + (jnp.arange(EP - E, dtype=jnp.int32) % (P - N))
    src2d = jnp.concatenate([src, pad_src]).reshape(EP // CHUNK, CHUNK)
    dst2d = jnp.concatenate([dst, pad_dst]).reshape(EP // CHUNK, CHUNK)

    x = jnp.zeros((P, DI), jnp.float32).at[:N].set(features)
    epsp = jnp.zeros((P, DO), jnp.float32).at[:N].set(eps)
    zer = jnp.zeros((ZROWS, DO), jnp.float32)

    cnt = _deg(src2d, dst2d)
    cs = cnt[0].reshape(NTILES, 128 * 128)[:, :P]
    cd = cnt[1].reshape(NTILES, 128 * 128)[:, :P]

    h0, ns, nd = _k1(x, W1, cs, cd)
    p1 = _prop(h0, src2d, dst2d, zer)
    h1 = _k2(p1, ns, nd, W2, b1.reshape(1, DO))
    p2 = _prop(h1, src2d, dst2d, zer)
    h2 = _k3(p2, ns, nd, b2.reshape(1, DO))
    p3 = _prop(h2, src2d, dst2d, zer)
    mu, sg, z = _k4(p3, nd, Wmu, bmu.reshape(1, DO), Wls, bls.reshape(1, DO),
                    epsp)
    return mu[:N], sg[:N], z[:N]
